# Initial kernel scaffold; baseline (speedup 1.0000x reference)
import jax
import jax.numpy as jnp
from jax.experimental import pallas as pl


def kernel(updates, mask):
    raise NotImplementedError("write your pallas kernel here")


# speedup vs baseline: 5.7726x; 5.7726x over previous
import functools

import jax
import jax.numpy as jnp
from jax import lax
from jax.experimental import pallas as pl
from jax.experimental.pallas import tpu as pltpu
from jax.experimental.pallas import tpu_sc as plsc

B, H, W_IN, C = 2, 192, 192, 96
OH, OW = 2 * H, 2 * W_IN
FLAT_OUT = OH * OW * C
T = B * H * W_IN * C
OTOT = B * FLAT_OUT

NCHUNK = 16
CH = OTOT // NCHUNK
NPASS = NCHUNK // 2
PT = T // 16
WIN = 8192
NWIN = PT // WIN
SL = CH // 16
ZB = SL // 36
DSTRIDE = CH // WIN

assert 16 * (2 * WIN + ZB) + CH <= 2_097_151

_mesh = plsc.VectorSubcoreMesh(core_axis_name="c", subcore_axis_name="s")


@functools.partial(
    pl.kernel,
    out_type=jax.ShapeDtypeStruct((OTOT,), jnp.float32),
    mesh=_mesh,
    scratch_types=[
        pltpu.VMEM((WIN,), jnp.int32),
        pltpu.VMEM((WIN,), jnp.float32),
        pltpu.VMEM((ZB,), jnp.float32),
        pltpu.VMEM_SHARED((CH,), jnp.float32),
    ],
)
def _scatter_kernel(idx_hbm, upd_hbm, out_hbm,
                    idxbuf, valbuf, zbuf, acc):
    c = lax.axis_index("c")
    s = lax.axis_index("s")
    boff = jnp.where(s < 8, jnp.int32(0), jnp.int32(FLAT_OUT))
    lanes = lax.iota(jnp.int32, 16)

    def zinit(i, carry):
        zbuf[pl.ds(i * 16, 16)] = jnp.zeros((16,), jnp.float32)
        return carry
    lax.fori_loop(0, ZB // 16, zinit, 0)

    def one_pass(p, carry):
        lo = (p * 2 + c) * CH

        def zero_slice(k, kcarry):
            pltpu.sync_copy(zbuf, acc.at[pl.ds(s * SL + k * ZB, ZB)])
            return kcarry
        lax.fori_loop(0, SL // ZB, zero_slice, 0)
        plsc.subcore_barrier()

        def one_window(w, wcarry):
            base = s * PT + w * WIN
            pltpu.sync_copy(idx_hbm.at[pl.ds(base, WIN)], idxbuf)
            pltpu.sync_copy(upd_hbm.at[pl.ds(base, WIN)], valbuf)

            def one_vec(i, vcarry):
                u = idxbuf[pl.ds(i * 16, 16)] + boff - lo
                v = valbuf[pl.ds(i * 16, 16)]
                m = (u >= 0) & (u < CH)
                pos = i * 16 + lanes
                idxbuf[pl.ds(i * 16, 16)] = jnp.where(m, u, pos * DSTRIDE)
                valbuf[pl.ds(i * 16, 16)] = jnp.where(m, v, jnp.float32(0.0))
                return vcarry
            lax.fori_loop(0, WIN // 16, one_vec, 0)
            pltpu.sync_copy(valbuf, acc.at[idxbuf], add=True)
            return wcarry
        lax.fori_loop(0, NWIN, one_window, 0)
        plsc.subcore_barrier()
        pltpu.sync_copy(acc.at[pl.ds(s * SL, SL)],
                        out_hbm.at[pl.ds(lo + s * SL, SL)])
        return carry
    lax.fori_loop(0, NPASS, one_pass, 0)


def kernel(updates, mask):
    idx_flat = mask.reshape(-1).astype(jnp.int32)
    upd_flat = updates.reshape(-1)
    out = _scatter_kernel(idx_flat, upd_flat)
    return out.reshape(B, OH, OW, C)
